# baseline (speedup 1.0000x reference)
import functools

import jax
import jax.numpy as jnp
from jax import lax
from jax.experimental import pallas as pl
from jax.experimental.pallas import tpu as pltpu
from jax.experimental.pallas import tpu_sc as plsc

B = 16384
D = 16
NC = 2
NS = 16
NW = NC * NS
B_PER_W = B // NW


def _build():
    mesh = plsc.VectorSubcoreMesh(core_axis_name="c", subcore_axis_name="s")

    @functools.partial(
        pl.kernel,
        mesh=mesh,
        out_type=jax.ShapeDtypeStruct((2, D, B), jnp.float32),
        compiler_params=pltpu.CompilerParams(use_tc_tiling_on_sc=False),
        scratch_types=[
            pltpu.VMEM((B_PER_W,), jnp.int32),
            pltpu.VMEM((B_PER_W,), jnp.int32),
            pltpu.VMEM((D, B_PER_W), jnp.float32),
            pltpu.VMEM((D, B_PER_W), jnp.float32),
            pltpu.SemaphoreType.DMA,
            pltpu.SemaphoreType.DMA,
        ],
    )
    def emb_lookup(x0_hbm, x1_hbm, ut_hbm, it_hbm, out_hbm,
                   idx_u, idx_i, dst_u, dst_i, sem_u, sem_i):
        wid = lax.axis_index("s") * NC + lax.axis_index("c")
        base = wid * B_PER_W
        pltpu.sync_copy(x0_hbm.at[pl.ds(base, B_PER_W)], idx_u)
        pltpu.sync_copy(x1_hbm.at[pl.ds(base, B_PER_W)], idx_i)
        cps = []
        for c in range(D):
            cps.append(pltpu.async_copy(
                ut_hbm.at[c].at[idx_u], dst_u.at[c], sem_u))
            cps.append(pltpu.async_copy(
                it_hbm.at[c].at[idx_i], dst_i.at[c], sem_i))
        for cp in cps:
            cp.wait()
        pltpu.sync_copy(dst_u, out_hbm.at[0, :, pl.ds(base, B_PER_W)])
        pltpu.sync_copy(dst_i, out_hbm.at[1, :, pl.ds(base, B_PER_W)])

    return emb_lookup


_emb_lookup = _build()


@jax.jit
def kernel(x, uid_table, iid_table):
    x0 = x[:, 0]
    x1 = x[:, 1]
    out_t = _emb_lookup(x0, x1, uid_table.T, iid_table.T)
    return out_t.transpose(2, 0, 1)


# speedup vs baseline: 11.1451x; 11.1451x over previous
import functools

import jax
import jax.numpy as jnp
from jax import lax
from jax.experimental import pallas as pl
from jax.experimental.pallas import tpu as pltpu
from jax.experimental.pallas import tpu_sc as plsc

B = 16384
D = 16
NC = 2
NS = 16
NW = NC * NS
B_PER_W = B // NW
LANES = 16


def _build():
    mesh = plsc.VectorSubcoreMesh(core_axis_name="c", subcore_axis_name="s")

    @functools.partial(
        pl.kernel,
        mesh=mesh,
        out_type=jax.ShapeDtypeStruct((2, D, B), jnp.float32),
        compiler_params=pltpu.CompilerParams(needs_layout_passes=False),
        scratch_types=[
            pltpu.VMEM((B_PER_W,), jnp.int32),
            pltpu.VMEM((B_PER_W,), jnp.int32),
            pltpu.VMEM((2, 2, D, 128), jnp.float32),
            pltpu.VMEM((D, B_PER_W), jnp.float32),
            pltpu.VMEM((D, B_PER_W), jnp.float32),
            pltpu.SemaphoreType.DMA,
            pltpu.SemaphoreType.DMA,
        ],
    )
    def emb_lookup(x0_hbm, x1_hbm, ut_hbm, it_hbm, out_hbm,
                   vidx_u, vidx_i, blk, dst_u, dst_i,
                   sem_u, sem_i):
        wid = lax.axis_index("s") * NC + lax.axis_index("c")
        base = wid * B_PER_W
        pltpu.sync_copy(x0_hbm.at[pl.ds(base, B_PER_W)], vidx_u)
        pltpu.sync_copy(x1_hbm.at[pl.ds(base, B_PER_W)], vidx_i)

        row_iota = lax.iota(jnp.int32, LANES)

        def offsets(k):
            kvec = jnp.full((LANES,), k, jnp.int32)
            iu = plsc.load_gather(vidx_u, [kvec])
            ii = plsc.load_gather(vidx_i, [kvec])
            off_u = pl.multiple_of(((iu >> 7) << 7)[0], 128)
            off_i = pl.multiple_of(((ii >> 7) << 7)[0], 128)
            return kvec, iu, ii, off_u, off_i

        def issue(k):
            par = lax.rem(k, 2)
            _, _, _, off_u, off_i = offsets(k)
            pltpu.async_copy(ut_hbm.at[:, pl.ds(off_u, 128)],
                             blk.at[0, par], sem_u)
            pltpu.async_copy(it_hbm.at[:, pl.ds(off_i, 128)],
                             blk.at[1, par], sem_i)

        def wait_and_extract(k):
            par = lax.rem(k, 2)
            kvec, iu, ii, off_u, off_i = offsets(k)
            pltpu.make_async_copy(ut_hbm.at[:, pl.ds(off_u, 128)],
                                  blk.at[0, par], sem_u).wait()
            pltpu.make_async_copy(it_hbm.at[:, pl.ds(off_i, 128)],
                                  blk.at[1, par], sem_i).wait()
            lane_u = iu & 127
            lane_i = ii & 127
            col_u = plsc.load_gather(blk.at[0, par], [row_iota, lane_u])
            col_i = plsc.load_gather(blk.at[1, par], [row_iota, lane_i])
            plsc.store_scatter(dst_u, [row_iota, kvec], col_u)
            plsc.store_scatter(dst_i, [row_iota, kvec], col_i)

        issue(0)

        def body(k, _):
            issue(k)
            wait_and_extract(k - 1)
            return 0

        lax.fori_loop(1, B_PER_W, body, 0)
        wait_and_extract(B_PER_W - 1)

        pltpu.sync_copy(dst_u, out_hbm.at[0, :, pl.ds(base, B_PER_W)])
        pltpu.sync_copy(dst_i, out_hbm.at[1, :, pl.ds(base, B_PER_W)])

    return emb_lookup


_emb_lookup = _build()


@jax.jit
def kernel(x, uid_table, iid_table):
    x0 = x[:, 0]
    x1 = x[:, 1]
    out_t = _emb_lookup(x0, x1, uid_table.T, iid_table.T)
    return out_t.transpose(2, 0, 1)


# speedup vs baseline: 21.2069x; 1.9028x over previous
import functools

import jax
import jax.numpy as jnp
from jax import lax
from jax.experimental import pallas as pl
from jax.experimental.pallas import tpu as pltpu
from jax.experimental.pallas import tpu_sc as plsc

B = 16384
D = 16
NC = 2
NS = 16
NW = NC * NS
B_PER_W = B // NW
LANES = 16


def _build():
    mesh = plsc.VectorSubcoreMesh(core_axis_name="c", subcore_axis_name="s")

    @functools.partial(
        pl.kernel,
        mesh=mesh,
        out_type=jax.ShapeDtypeStruct((2, D, B), jnp.float32),
        compiler_params=pltpu.CompilerParams(needs_layout_passes=False),
        scratch_types=[
            pltpu.VMEM((B_PER_W,), jnp.int32),
            pltpu.VMEM((B_PER_W,), jnp.int32),
            pltpu.VMEM((2, 8, D, 128), jnp.float32),
            pltpu.VMEM((D, B_PER_W), jnp.float32),
            pltpu.VMEM((D, B_PER_W), jnp.float32),
            pltpu.SemaphoreType.DMA,
            pltpu.SemaphoreType.DMA,
        ],
    )
    def emb_lookup(x0_hbm, x1_hbm, ut_hbm, it_hbm, out_hbm,
                   vidx_u, vidx_i, blk, dst_u, dst_i,
                   sem_u, sem_i):
        wid = lax.axis_index("s") * NC + lax.axis_index("c")
        base = wid * B_PER_W
        pltpu.sync_copy(x0_hbm.at[pl.ds(base, B_PER_W)], vidx_u)
        pltpu.sync_copy(x1_hbm.at[pl.ds(base, B_PER_W)], vidx_i)

        row_iota = lax.iota(jnp.int32, LANES)

        def offsets(k):
            kvec = jnp.full((LANES,), k, jnp.int32)
            iu = plsc.load_gather(vidx_u, [kvec])
            ii = plsc.load_gather(vidx_i, [kvec])
            off_u = pl.multiple_of(((iu >> 7) << 7)[0], 128)
            off_i = pl.multiple_of(((ii >> 7) << 7)[0], 128)
            return kvec, iu, ii, off_u, off_i

        DEPTH = 8

        def issue(k):
            par = lax.rem(k, DEPTH)
            _, _, _, off_u, off_i = offsets(k)
            pltpu.async_copy(ut_hbm.at[:, pl.ds(off_u, 128)],
                             blk.at[0, par], sem_u)
            pltpu.async_copy(it_hbm.at[:, pl.ds(off_i, 128)],
                             blk.at[1, par], sem_i)

        def wait_and_extract(k):
            par = lax.rem(k, DEPTH)
            kvec, iu, ii, off_u, off_i = offsets(k)
            pltpu.make_async_copy(ut_hbm.at[:, pl.ds(off_u, 128)],
                                  blk.at[0, par], sem_u).wait()
            pltpu.make_async_copy(it_hbm.at[:, pl.ds(off_i, 128)],
                                  blk.at[1, par], sem_i).wait()
            lane_u = iu & 127
            lane_i = ii & 127
            col_u = plsc.load_gather(blk.at[0, par], [row_iota, lane_u])
            col_i = plsc.load_gather(blk.at[1, par], [row_iota, lane_i])
            plsc.store_scatter(dst_u, [row_iota, kvec], col_u)
            plsc.store_scatter(dst_i, [row_iota, kvec], col_i)

        def prologue(k, _):
            issue(k)
            return 0

        lax.fori_loop(0, 7, prologue, 0)

        def body(k, _):
            issue(k)
            wait_and_extract(k - 7)
            return 0

        lax.fori_loop(7, B_PER_W, body, 0)

        def epilogue(k, _):
            wait_and_extract(k)
            return 0

        lax.fori_loop(B_PER_W - 7, B_PER_W, epilogue, 0)

        pltpu.sync_copy(dst_u, out_hbm.at[0, :, pl.ds(base, B_PER_W)])
        pltpu.sync_copy(dst_i, out_hbm.at[1, :, pl.ds(base, B_PER_W)])

    return emb_lookup


_emb_lookup = _build()


@jax.jit
def kernel(x, uid_table, iid_table):
    x0 = x[:, 0]
    x1 = x[:, 1]
    out_t = _emb_lookup(x0, x1, uid_table.T, iid_table.T)
    return out_t.transpose(2, 0, 1)


# speedup vs baseline: 23.1540x; 1.0918x over previous
import functools

import jax
import jax.numpy as jnp
from jax import lax
from jax.experimental import pallas as pl
from jax.experimental.pallas import tpu as pltpu
from jax.experimental.pallas import tpu_sc as plsc

B = 16384
D = 16
NC = 2
NS = 16
NW = NC * NS
B_PER_W = B // NW
LANES = 16


def _build():
    mesh = plsc.VectorSubcoreMesh(core_axis_name="c", subcore_axis_name="s")

    @functools.partial(
        pl.kernel,
        mesh=mesh,
        out_type=jax.ShapeDtypeStruct((2, D, B), jnp.float32),
        compiler_params=pltpu.CompilerParams(needs_layout_passes=False),
        scratch_types=[
            pltpu.VMEM((B_PER_W,), jnp.int32),
            pltpu.VMEM((B_PER_W,), jnp.int32),
            pltpu.VMEM((2, 16, D, 128), jnp.float32),
            pltpu.VMEM((D, B_PER_W), jnp.float32),
            pltpu.VMEM((D, B_PER_W), jnp.float32),
            pltpu.SemaphoreType.DMA,
            pltpu.SemaphoreType.DMA,
        ],
    )
    def emb_lookup(x0_hbm, x1_hbm, ut_hbm, it_hbm, out_hbm,
                   vidx_u, vidx_i, blk, dst_u, dst_i,
                   sem_u, sem_i):
        wid = lax.axis_index("s") * NC + lax.axis_index("c")
        base = wid * B_PER_W
        pltpu.sync_copy(x0_hbm.at[pl.ds(base, B_PER_W)], vidx_u)
        pltpu.sync_copy(x1_hbm.at[pl.ds(base, B_PER_W)], vidx_i)

        row_iota = lax.iota(jnp.int32, LANES)

        def offsets(k):
            kvec = jnp.full((LANES,), k, jnp.int32)
            iu = plsc.load_gather(vidx_u, [kvec])
            ii = plsc.load_gather(vidx_i, [kvec])
            off_u = pl.multiple_of(((iu >> 7) << 7)[0], 128)
            off_i = pl.multiple_of(((ii >> 7) << 7)[0], 128)
            return kvec, iu, ii, off_u, off_i

        DEPTH = 16

        def issue(k):
            par = lax.rem(k, DEPTH)
            _, _, _, off_u, off_i = offsets(k)
            pltpu.async_copy(ut_hbm.at[:, pl.ds(off_u, 128)],
                             blk.at[0, par], sem_u)
            pltpu.async_copy(it_hbm.at[:, pl.ds(off_i, 128)],
                             blk.at[1, par], sem_i)

        def wait_and_extract(k):
            par = lax.rem(k, DEPTH)
            kvec, iu, ii, off_u, off_i = offsets(k)
            pltpu.make_async_copy(ut_hbm.at[:, pl.ds(off_u, 128)],
                                  blk.at[0, par], sem_u).wait()
            pltpu.make_async_copy(it_hbm.at[:, pl.ds(off_i, 128)],
                                  blk.at[1, par], sem_i).wait()
            lane_u = iu & 127
            lane_i = ii & 127
            col_u = plsc.load_gather(blk.at[0, par], [row_iota, lane_u])
            col_i = plsc.load_gather(blk.at[1, par], [row_iota, lane_i])
            plsc.store_scatter(dst_u, [row_iota, kvec], col_u)
            plsc.store_scatter(dst_i, [row_iota, kvec], col_i)

        def prologue(k, _):
            issue(k)
            return 0

        lax.fori_loop(0, 15, prologue, 0)

        def body(k, _):
            issue(k)
            wait_and_extract(k - 15)
            return 0

        lax.fori_loop(15, B_PER_W, body, 0)

        def epilogue(k, _):
            wait_and_extract(k)
            return 0

        lax.fori_loop(B_PER_W - 15, B_PER_W, epilogue, 0)

        pltpu.sync_copy(dst_u, out_hbm.at[0, :, pl.ds(base, B_PER_W)])
        pltpu.sync_copy(dst_i, out_hbm.at[1, :, pl.ds(base, B_PER_W)])

    return emb_lookup


_emb_lookup = _build()


@jax.jit
def kernel(x, uid_table, iid_table):
    x0 = x[:, 0]
    x1 = x[:, 1]
    out_t = _emb_lookup(x0, x1, uid_table.T, iid_table.T)
    return out_t.transpose(2, 0, 1)
